# Initial kernel scaffold; baseline (speedup 1.0000x reference)
import numpy as np
import jax
import jax.numpy as jnp
from jax import lax
from jax.experimental import pallas as pl
from jax.experimental.pallas import tpu as pltpu
from jax.experimental.pallas import tpu_sc as plsc

_N = 10000
_E = 320000
_H = 128
_NLAYERS = 4
_FREQ = 256

_NB = _E // 128
_WORKERS = 32
_BPW = _NB // _WORKERS
_EXTRA = _NB - _BPW * _WORKERS

_NBLK = 1000
_EBLK = 3200

_f32 = jnp.float32


def _silu(v):
    return v * jax.nn.sigmoid(v)


def _init_body(tf_ref, wt1, bt1, wt2, bt2, h_ref, wemb, bemb, we1a, we1b, be1,
               hh_o, prow_o, pcol_o, temb_o):
    te = _silu(tf_ref[...] @ wt1[...] + bt1[...]) @ wt2[...] + bt2[...]
    temb_o[...] = te
    hh = h_ref[...] @ wemb[...] + bemb[...] + te[0:1, :]
    hh_o[...] = hh
    prow_o[...] = hh @ we1a[...] + be1[...]
    pcol_o[...] = hh @ we1b[...]


def _edge_body(grow, gcol, xr, xc, ea, w1r, w1e, we2, be2, wc1, bc1, wc2,
               ef_o, tr_o):
    d = xr[...] - xc[...]
    radial = jnp.sum(d * d, axis=1, keepdims=True)
    norm = jnp.sqrt(radial + 1e-8)
    cd = d / (norm + 1.0)
    m = _silu(grow[...] + gcol[...] + radial * w1r[...] + ea[...] @ w1e[...])
    ef = _silu(m @ we2[...] + be2[...])
    cm = _silu(ef @ wc1[...] + bc1[...])
    s = jnp.sum(cm * wc2[...], axis=1, keepdims=True)
    ef_o[...] = ef
    tr_o[...] = cd * s


def _node_body(hh_ref, ph0, ph1, px0, px1, coord_ref, temb_ref,
               wn1a, wn1b, bn1, wn2, bn2, we1a, we1b, be1,
               hh_o, coord_o, prow_o, pcol_o):
    hh = hh_ref[...]
    aggh = ph0[...] + ph1[...]
    o = _silu(hh @ wn1a[...] + aggh @ wn1b[...] + bn1[...]) @ wn2[...] + bn2[...]
    hhn = hh + o + temb_ref[0:1, :]
    hh_o[...] = hhn
    coord_o[...] = coord_ref[...] + px0[...] + px1[...]
    prow_o[...] = hhn @ we1a[...] + be1[...]
    pcol_o[...] = hhn @ we1b[...]


def _final_body(hh_ref, ph0, ph1, px0, px1, coord_ref,
                wn1a, wn1b, bn1, wn2, bn2, wout, bout,
                hout_o, coord_o):
    hh = hh_ref[...]
    aggh = ph0[...] + ph1[...]
    o = _silu(hh @ wn1a[...] + aggh @ wn1b[...] + bn1[...]) @ wn2[...] + bn2[...]
    hhn = hh + o
    hout_o[...] = hhn @ wout[...] + bout[...]
    coord_o[...] = coord_ref[...] + px0[...] + px1[...]


_MESH = plsc.VectorSubcoreMesh(core_axis_name="c", subcore_axis_name="s")


def _gather_body(prow, pcol, xp, row2, col2,
                 grow_o, gcol_o, xr_o, xc_o,
                 idxr, idxc, grow_v, gcol_v, xr_v, xc_v, s0, s1, s2, s3):
    cid = lax.axis_index("c")
    sid = lax.axis_index("s")
    wid = sid * 2 + cid
    pltpu.sync_copy(row2.at[pl.ds(wid * _BPW, _BPW)], idxr)
    pltpu.sync_copy(col2.at[pl.ds(wid * _BPW, _BPW)], idxc)

    def do_block(j, b):
        d0 = pltpu.async_copy(prow.at[idxr.at[j]], grow_v, s0)
        d1 = pltpu.async_copy(pcol.at[idxc.at[j]], gcol_v, s1)
        d2 = pltpu.async_copy(xp.at[idxr.at[j]], xr_v, s2)
        d3 = pltpu.async_copy(xp.at[idxc.at[j]], xc_v, s3)
        d0.wait()
        d1.wait()
        d2.wait()
        d3.wait()
        base = b * 128
        pltpu.sync_copy(grow_v, grow_o.at[pl.ds(base, 128)])
        pltpu.sync_copy(gcol_v, gcol_o.at[pl.ds(base, 128)])
        pltpu.sync_copy(xr_v, xr_o.at[pl.ds(base, 128)])
        pltpu.sync_copy(xc_v, xc_o.at[pl.ds(base, 128)])

    def loop(j, carry):
        do_block(j, wid * _BPW + j)
        return carry
    lax.fori_loop(0, _BPW, loop, 0)

    @pl.when(wid < _EXTRA)
    def _():
        b = _WORKERS * _BPW + wid
        pltpu.sync_copy(row2.at[pl.ds(b, 1)], idxr.at[pl.ds(0, 1)])
        pltpu.sync_copy(col2.at[pl.ds(b, 1)], idxc.at[pl.ds(0, 1)])
        do_block(0, b)


def _scatter_body(ef, tr, row2, zh, zx, ph_o, px_o,
                  idxb, ef_v, tr_v, sh, sx):
    cid = lax.axis_index("c")
    sid = lax.axis_index("s")
    wid = sid * 2 + cid

    @pl.when(sid == 0)
    def _():
        pltpu.sync_copy(zh, sh)
        pltpu.sync_copy(zx, sx)
    plsc.subcore_barrier()

    def do_block(b):
        pltpu.sync_copy(row2.at[pl.ds(b, 1)], idxb)
        pltpu.sync_copy(ef.at[pl.ds(b * 128, 128)], ef_v)
        pltpu.sync_copy(tr.at[pl.ds(b * 128, 128)], tr_v)
        pltpu.sync_copy(ef_v, sh.at[idxb.at[0]], add=True)
        pltpu.sync_copy(tr_v, sx.at[idxb.at[0]], add=True)

    def loop(j, carry):
        do_block(wid * _BPW + j)
        return carry
    lax.fori_loop(0, _BPW, loop, 0)

    @pl.when(wid < _EXTRA)
    def _():
        do_block(_WORKERS * _BPW + wid)

    plsc.subcore_barrier()

    @pl.when(sid == 0)
    def _():
        pltpu.sync_copy(sh, ph_o.at[cid])
        pltpu.sync_copy(sx, px_o.at[cid])


def _make_sc_gather():
    return pl.kernel(
        _gather_body,
        out_type=(
            jax.ShapeDtypeStruct((_E, _H), _f32),
            jax.ShapeDtypeStruct((_E, _H), _f32),
            jax.ShapeDtypeStruct((_E, 8), _f32),
            jax.ShapeDtypeStruct((_E, 8), _f32),
        ),
        mesh=_MESH,
        scratch_types=[
            pltpu.VMEM((_BPW, 128), jnp.int32),
            pltpu.VMEM((_BPW, 128), jnp.int32),
            pltpu.VMEM((128, _H), _f32),
            pltpu.VMEM((128, _H), _f32),
            pltpu.VMEM((128, 8), _f32),
            pltpu.VMEM((128, 8), _f32),
            pltpu.SemaphoreType.DMA,
            pltpu.SemaphoreType.DMA,
            pltpu.SemaphoreType.DMA,
            pltpu.SemaphoreType.DMA,
        ],
    )


def _make_sc_scatter():
    return pl.kernel(
        _scatter_body,
        out_type=(
            jax.ShapeDtypeStruct((2, _N, _H), _f32),
            jax.ShapeDtypeStruct((2, _N, 8), _f32),
        ),
        mesh=_MESH,
        scratch_types=[
            pltpu.VMEM((1, 128), jnp.int32),
            pltpu.VMEM((128, _H), _f32),
            pltpu.VMEM((128, 8), _f32),
            pltpu.VMEM_SHARED((_N, _H), _f32),
            pltpu.VMEM_SHARED((_N, 8), _f32),
        ],
    )


def _bs(shape, const=False):
    if const:
        return pl.BlockSpec(shape, lambda i: (0, 0))
    return pl.BlockSpec(shape, lambda i: (i, 0))


def _make_init():
    n = _N // _NBLK
    return pl.pallas_call(
        _init_body,
        grid=(n,),
        in_specs=[
            _bs((8, _FREQ), True), _bs((_FREQ, _H), True), _bs((1, _H), True),
            _bs((_H, _H), True), _bs((1, _H), True),
            _bs((_NBLK, _H)),
            _bs((_H, _H), True), _bs((1, _H), True),
            _bs((_H, _H), True), _bs((_H, _H), True), _bs((1, _H), True),
        ],
        out_specs=[
            _bs((_NBLK, _H)), _bs((_NBLK, _H)), _bs((_NBLK, _H)),
            _bs((8, _H), True),
        ],
        out_shape=[
            jax.ShapeDtypeStruct((_N, _H), _f32),
            jax.ShapeDtypeStruct((_N, _H), _f32),
            jax.ShapeDtypeStruct((_N, _H), _f32),
            jax.ShapeDtypeStruct((8, _H), _f32),
        ],
    )


def _make_edge():
    n = _E // _EBLK
    return pl.pallas_call(
        _edge_body,
        grid=(n,),
        in_specs=[
            _bs((_EBLK, _H)), _bs((_EBLK, _H)),
            _bs((_EBLK, 8)), _bs((_EBLK, 8)), _bs((_EBLK, 8)),
            _bs((1, _H), True), _bs((8, _H), True),
            _bs((_H, _H), True), _bs((1, _H), True),
            _bs((_H, _H), True), _bs((1, _H), True), _bs((1, _H), True),
        ],
        out_specs=[_bs((_EBLK, _H)), _bs((_EBLK, 8))],
        out_shape=[
            jax.ShapeDtypeStruct((_E, _H), _f32),
            jax.ShapeDtypeStruct((_E, 8), _f32),
        ],
    )


def _make_node():
    n = _N // _NBLK
    return pl.pallas_call(
        _node_body,
        grid=(n,),
        in_specs=[
            _bs((_NBLK, _H)),
            _bs((_NBLK, _H)), _bs((_NBLK, _H)),
            _bs((_NBLK, 8)), _bs((_NBLK, 8)),
            _bs((_NBLK, 8)),
            _bs((8, _H), True),
            _bs((_H, _H), True), _bs((_H, _H), True), _bs((1, _H), True),
            _bs((_H, _H), True), _bs((1, _H), True),
            _bs((_H, _H), True), _bs((_H, _H), True), _bs((1, _H), True),
        ],
        out_specs=[
            _bs((_NBLK, _H)), _bs((_NBLK, 8)),
            _bs((_NBLK, _H)), _bs((_NBLK, _H)),
        ],
        out_shape=[
            jax.ShapeDtypeStruct((_N, _H), _f32),
            jax.ShapeDtypeStruct((_N, 8), _f32),
            jax.ShapeDtypeStruct((_N, _H), _f32),
            jax.ShapeDtypeStruct((_N, _H), _f32),
        ],
    )


def _make_final():
    n = _N // _NBLK
    return pl.pallas_call(
        _final_body,
        grid=(n,),
        in_specs=[
            _bs((_NBLK, _H)),
            _bs((_NBLK, _H)), _bs((_NBLK, _H)),
            _bs((_NBLK, 8)), _bs((_NBLK, 8)),
            _bs((_NBLK, 8)),
            _bs((_H, _H), True), _bs((_H, _H), True), _bs((1, _H), True),
            _bs((_H, _H), True), _bs((1, _H), True),
            _bs((_H, _H), True), _bs((1, _H), True),
        ],
        out_specs=[_bs((_NBLK, _H)), _bs((_NBLK, 8))],
        out_shape=[
            jax.ShapeDtypeStruct((_N, _H), _f32),
            jax.ShapeDtypeStruct((_N, 8), _f32),
        ],
    )


def kernel(h, x, t, edges, edge_attr, params):
    p = params

    half = _FREQ // 2
    freqs = jnp.exp(-np.log(10000.0) * jnp.arange(half, dtype=_f32) / half)
    args = t.astype(_f32)[:, None] * freqs[None]
    tf = jnp.concatenate([jnp.cos(args), jnp.sin(args)], axis=-1)
    tf8 = jnp.broadcast_to(tf, (8, _FREQ))

    xp = jnp.pad(x.astype(_f32), ((0, 0), (0, 5)))
    row2 = edges[0].reshape(_NB, 128)
    col2 = edges[1].reshape(_NB, 128)
    ea = jnp.pad(edge_attr.astype(_f32), ((0, 0), (0, 4)))
    zh = jnp.zeros((_N, _H), _f32)
    zx = jnp.zeros((_N, 8), _f32)

    init = _make_init()
    edgek = _make_edge()
    nodek = _make_node()
    finalk = _make_final()
    gath = _make_sc_gather()
    scat = _make_sc_scatter()

    we1 = p['We1'].astype(_f32)
    wn1 = p['Wn1'].astype(_f32)

    hh, prow, pcol, temb = init(
        tf8, p['Wt1'], p['bt1'].reshape(1, _H), p['Wt2'], p['bt2'].reshape(1, _H),
        h.astype(_f32), p['W_emb'], p['b_emb'].reshape(1, _H),
        we1[0, :_H], we1[0, _H:2 * _H], p['be1'][0].reshape(1, _H))

    coord = xp
    h_out = None
    for i in range(_NLAYERS):
        w1r = we1[i, 2 * _H:2 * _H + 1]
        w1e = jnp.pad(we1[i, 2 * _H + 1:], ((0, 4), (0, 0)))
        grow, gcol, xr, xc = gath(prow, pcol, coord, row2, col2)
        ef, tr = edgek(grow, gcol, xr, xc, ea, w1r, w1e,
                       p['We2'][i], p['be2'][i].reshape(1, _H),
                       p['Wc1'][i], p['bc1'][i].reshape(1, _H),
                       p['Wc2'][i].reshape(1, _H))
        ph, px = scat(ef, tr, row2, zh, zx)
        if i < _NLAYERS - 1:
            hh, coord, prow, pcol = nodek(
                hh, ph[0], ph[1], px[0], px[1], coord, temb,
                wn1[i, :_H], wn1[i, _H:], p['bn1'][i].reshape(1, _H),
                p['Wn2'][i], p['bn2'][i].reshape(1, _H),
                we1[i + 1, :_H], we1[i + 1, _H:2 * _H],
                p['be1'][i + 1].reshape(1, _H))
        else:
            h_out, coord = finalk(
                hh, ph[0], ph[1], px[0], px[1], coord,
                wn1[i, :_H], wn1[i, _H:], p['bn1'][i].reshape(1, _H),
                p['Wn2'][i], p['bn2'][i].reshape(1, _H),
                p['W_out'], p['b_out'].reshape(1, _H))

    return h_out, coord[:, :3]


# speedup vs baseline: 2.9255x; 2.9255x over previous
import numpy as np
import jax
import jax.numpy as jnp
from jax import lax
from jax.experimental import pallas as pl
from jax.experimental.pallas import tpu as pltpu
from jax.experimental.pallas import tpu_sc as plsc

_N = 10000
_E = 320000
_H = 128
_NLAYERS = 4
_FREQ = 256

_NB = _E // 128
_WORKERS = 32
_BPW = _NB // _WORKERS
_EXTRA = _NB - _BPW * _WORKERS

_NBLK = 1000
_EBLK = 3200

_f32 = jnp.float32


def _silu(v):
    return v * jax.nn.sigmoid(v)


def _init_body(tf_ref, wt1, bt1, wt2, bt2, h_ref, wemb, bemb, we1a, we1b, be1,
               hh_o, prow_o, pcol_o, temb_o):
    te = _silu(tf_ref[...] @ wt1[...] + bt1[...]) @ wt2[...] + bt2[...]
    temb_o[...] = te
    hh = h_ref[...] @ wemb[...] + bemb[...] + te[0:1, :]
    hh_o[...] = hh
    prow_o[...] = hh @ we1a[...] + be1[...]
    pcol_o[...] = hh @ we1b[...]


def _edge_body(grow, gcol, xr, xc, ea, w1r, w1e, we2, be2, wc1, bc1, wc2,
               ef_o, tr_o):
    d = xr[...] - xc[...]
    radial = jnp.sum(d * d, axis=1, keepdims=True)
    norm = jnp.sqrt(radial + 1e-8)
    cd = d / (norm + 1.0)
    m = _silu(grow[...] + gcol[...] + radial * w1r[...] + ea[...] @ w1e[...])
    ef = _silu(m @ we2[...] + be2[...])
    cm = _silu(ef @ wc1[...] + bc1[...])
    s = jnp.sum(cm * wc2[...], axis=1, keepdims=True)
    ef_o[...] = ef
    tr_o[...] = cd * s


def _node_body(hh_ref, ph0, ph1, px0, px1, coord_ref, temb_ref,
               wn1a, wn1b, bn1, wn2, bn2, we1a, we1b, be1,
               hh_o, coord_o, prow_o, pcol_o):
    hh = hh_ref[...]
    aggh = ph0[...] + ph1[...]
    o = _silu(hh @ wn1a[...] + aggh @ wn1b[...] + bn1[...]) @ wn2[...] + bn2[...]
    hhn = hh + o + temb_ref[0:1, :]
    hh_o[...] = hhn
    coord_o[...] = coord_ref[...] + px0[...] + px1[...]
    prow_o[...] = hhn @ we1a[...] + be1[...]
    pcol_o[...] = hhn @ we1b[...]


def _final_body(hh_ref, ph0, ph1, px0, px1, coord_ref,
                wn1a, wn1b, bn1, wn2, bn2, wout, bout,
                hout_o, coord_o):
    hh = hh_ref[...]
    aggh = ph0[...] + ph1[...]
    o = _silu(hh @ wn1a[...] + aggh @ wn1b[...] + bn1[...]) @ wn2[...] + bn2[...]
    hhn = hh + o
    hout_o[...] = hhn @ wout[...] + bout[...]
    coord_o[...] = coord_ref[...] + px0[...] + px1[...]


_MESH = plsc.VectorSubcoreMesh(core_axis_name="c", subcore_axis_name="s")


def _gather_body(prow, pcol, xp, row1, col1,
                 grow_o, gcol_o, xr_o, xc_o,
                 idxr, idxc, grow_v, gcol_v, xr_v, xc_v, s0, s1, s2, s3):
    cid = lax.axis_index("c")
    sid = lax.axis_index("s")
    wid = sid * 2 + cid

    def do_block(b):
        base = b * 128
        pltpu.sync_copy(row1.at[pl.ds(base, 128)], idxr)
        pltpu.sync_copy(col1.at[pl.ds(base, 128)], idxc)
        d0 = pltpu.async_copy(prow.at[idxr], grow_v, s0)
        d1 = pltpu.async_copy(pcol.at[idxc], gcol_v, s1)
        d2 = pltpu.async_copy(xp.at[idxr], xr_v, s2)
        d3 = pltpu.async_copy(xp.at[idxc], xc_v, s3)
        d0.wait()
        d1.wait()
        d2.wait()
        d3.wait()
        pltpu.sync_copy(grow_v, grow_o.at[pl.ds(base, 128)])
        pltpu.sync_copy(gcol_v, gcol_o.at[pl.ds(base, 128)])
        pltpu.sync_copy(xr_v, xr_o.at[pl.ds(base, 128)])
        pltpu.sync_copy(xc_v, xc_o.at[pl.ds(base, 128)])

    def loop(j, carry):
        do_block(wid * _BPW + j)
        return carry
    lax.fori_loop(0, _BPW, loop, 0)

    @pl.when(wid < _EXTRA)
    def _():
        do_block(_WORKERS * _BPW + wid)


def _scatter_body(ef, tr, row1, zh, zx, ph_o, px_o,
                  idxb, ef_v, tr_v, sh, sx):
    cid = lax.axis_index("c")
    sid = lax.axis_index("s")
    wid = sid * 2 + cid

    @pl.when(sid == 0)
    def _():
        pltpu.sync_copy(zh, sh)
        pltpu.sync_copy(zx, sx)
    plsc.subcore_barrier()

    def do_block(b):
        pltpu.sync_copy(row1.at[pl.ds(b * 128, 128)], idxb)
        pltpu.sync_copy(ef.at[pl.ds(b * 128, 128)], ef_v)
        pltpu.sync_copy(tr.at[pl.ds(b * 128, 128)], tr_v)
        pltpu.sync_copy(ef_v, sh.at[idxb], add=True)
        pltpu.sync_copy(tr_v, sx.at[idxb], add=True)

    def loop(j, carry):
        do_block(wid * _BPW + j)
        return carry
    lax.fori_loop(0, _BPW, loop, 0)

    @pl.when(wid < _EXTRA)
    def _():
        do_block(_WORKERS * _BPW + wid)

    plsc.subcore_barrier()

    @pl.when(sid == 0)
    def _():
        pltpu.sync_copy(sh, ph_o.at[cid])
        pltpu.sync_copy(sx, px_o.at[cid])


def _make_sc_gather():
    return pl.kernel(
        _gather_body,
        out_type=(
            jax.ShapeDtypeStruct((_E, _H), _f32),
            jax.ShapeDtypeStruct((_E, _H), _f32),
            jax.ShapeDtypeStruct((_E, 8), _f32),
            jax.ShapeDtypeStruct((_E, 8), _f32),
        ),
        mesh=_MESH,
        scratch_types=[
            pltpu.VMEM((128,), jnp.int32),
            pltpu.VMEM((128,), jnp.int32),
            pltpu.VMEM((128, _H), _f32),
            pltpu.VMEM((128, _H), _f32),
            pltpu.VMEM((128, 8), _f32),
            pltpu.VMEM((128, 8), _f32),
            pltpu.SemaphoreType.DMA,
            pltpu.SemaphoreType.DMA,
            pltpu.SemaphoreType.DMA,
            pltpu.SemaphoreType.DMA,
        ],
        compiler_params=pltpu.CompilerParams(use_tc_tiling_on_sc=False),
    )


def _make_sc_scatter():
    return pl.kernel(
        _scatter_body,
        out_type=(
            jax.ShapeDtypeStruct((2, _N, _H), _f32),
            jax.ShapeDtypeStruct((2, _N, 8), _f32),
        ),
        mesh=_MESH,
        scratch_types=[
            pltpu.VMEM((128,), jnp.int32),
            pltpu.VMEM((128, _H), _f32),
            pltpu.VMEM((128, 8), _f32),
            pltpu.VMEM_SHARED((_N, _H), _f32),
            pltpu.VMEM_SHARED((_N, 8), _f32),
        ],
        compiler_params=pltpu.CompilerParams(use_tc_tiling_on_sc=False),
    )


def _bs(shape, const=False):
    if const:
        return pl.BlockSpec(shape, lambda i: (0, 0))
    return pl.BlockSpec(shape, lambda i: (i, 0))


def _make_init():
    n = _N // _NBLK
    return pl.pallas_call(
        _init_body,
        grid=(n,),
        in_specs=[
            _bs((8, _FREQ), True), _bs((_FREQ, _H), True), _bs((1, _H), True),
            _bs((_H, _H), True), _bs((1, _H), True),
            _bs((_NBLK, _H)),
            _bs((_H, _H), True), _bs((1, _H), True),
            _bs((_H, _H), True), _bs((_H, _H), True), _bs((1, _H), True),
        ],
        out_specs=[
            _bs((_NBLK, _H)), _bs((_NBLK, _H)), _bs((_NBLK, _H)),
            _bs((8, _H), True),
        ],
        out_shape=[
            jax.ShapeDtypeStruct((_N, _H), _f32),
            jax.ShapeDtypeStruct((_N, _H), _f32),
            jax.ShapeDtypeStruct((_N, _H), _f32),
            jax.ShapeDtypeStruct((8, _H), _f32),
        ],
    )


def _make_edge():
    n = _E // _EBLK
    return pl.pallas_call(
        _edge_body,
        grid=(n,),
        in_specs=[
            _bs((_EBLK, _H)), _bs((_EBLK, _H)),
            _bs((_EBLK, 8)), _bs((_EBLK, 8)), _bs((_EBLK, 8)),
            _bs((1, _H), True), _bs((8, _H), True),
            _bs((_H, _H), True), _bs((1, _H), True),
            _bs((_H, _H), True), _bs((1, _H), True), _bs((1, _H), True),
        ],
        out_specs=[_bs((_EBLK, _H)), _bs((_EBLK, 8))],
        out_shape=[
            jax.ShapeDtypeStruct((_E, _H), _f32),
            jax.ShapeDtypeStruct((_E, 8), _f32),
        ],
    )


def _make_node():
    n = _N // _NBLK
    return pl.pallas_call(
        _node_body,
        grid=(n,),
        in_specs=[
            _bs((_NBLK, _H)),
            _bs((_NBLK, _H)), _bs((_NBLK, _H)),
            _bs((_NBLK, 8)), _bs((_NBLK, 8)),
            _bs((_NBLK, 8)),
            _bs((8, _H), True),
            _bs((_H, _H), True), _bs((_H, _H), True), _bs((1, _H), True),
            _bs((_H, _H), True), _bs((1, _H), True),
            _bs((_H, _H), True), _bs((_H, _H), True), _bs((1, _H), True),
        ],
        out_specs=[
            _bs((_NBLK, _H)), _bs((_NBLK, 8)),
            _bs((_NBLK, _H)), _bs((_NBLK, _H)),
        ],
        out_shape=[
            jax.ShapeDtypeStruct((_N, _H), _f32),
            jax.ShapeDtypeStruct((_N, 8), _f32),
            jax.ShapeDtypeStruct((_N, _H), _f32),
            jax.ShapeDtypeStruct((_N, _H), _f32),
        ],
    )


def _make_final():
    n = _N // _NBLK
    return pl.pallas_call(
        _final_body,
        grid=(n,),
        in_specs=[
            _bs((_NBLK, _H)),
            _bs((_NBLK, _H)), _bs((_NBLK, _H)),
            _bs((_NBLK, 8)), _bs((_NBLK, 8)),
            _bs((_NBLK, 8)),
            _bs((_H, _H), True), _bs((_H, _H), True), _bs((1, _H), True),
            _bs((_H, _H), True), _bs((1, _H), True),
            _bs((_H, _H), True), _bs((1, _H), True),
        ],
        out_specs=[_bs((_NBLK, _H)), _bs((_NBLK, 8))],
        out_shape=[
            jax.ShapeDtypeStruct((_N, _H), _f32),
            jax.ShapeDtypeStruct((_N, 8), _f32),
        ],
    )


def kernel(h, x, t, edges, edge_attr, params):
    p = params

    half = _FREQ // 2
    freqs = jnp.exp(-np.log(10000.0) * jnp.arange(half, dtype=_f32) / half)
    args = t.astype(_f32)[:, None] * freqs[None]
    tf = jnp.concatenate([jnp.cos(args), jnp.sin(args)], axis=-1)
    tf8 = jnp.broadcast_to(tf, (8, _FREQ))

    xp = jnp.pad(x.astype(_f32), ((0, 0), (0, 5)))
    row1 = edges[0]
    col1 = edges[1]
    ea = jnp.pad(edge_attr.astype(_f32), ((0, 0), (0, 4)))
    zh = jnp.zeros((_N, _H), _f32)
    zx = jnp.zeros((_N, 8), _f32)

    init = _make_init()
    edgek = _make_edge()
    nodek = _make_node()
    finalk = _make_final()
    gath = _make_sc_gather()
    scat = _make_sc_scatter()

    we1 = p['We1'].astype(_f32)
    wn1 = p['Wn1'].astype(_f32)

    hh, prow, pcol, temb = init(
        tf8, p['Wt1'], p['bt1'].reshape(1, _H), p['Wt2'], p['bt2'].reshape(1, _H),
        h.astype(_f32), p['W_emb'], p['b_emb'].reshape(1, _H),
        we1[0, :_H], we1[0, _H:2 * _H], p['be1'][0].reshape(1, _H))

    coord = xp
    h_out = None
    for i in range(_NLAYERS):
        w1r = we1[i, 2 * _H:2 * _H + 1]
        w1e = jnp.pad(we1[i, 2 * _H + 1:], ((0, 4), (0, 0)))
        grow, gcol, xr, xc = gath(prow, pcol, coord, row1, col1)
        ef, tr = edgek(grow, gcol, xr, xc, ea, w1r, w1e,
                       p['We2'][i], p['be2'][i].reshape(1, _H),
                       p['Wc1'][i], p['bc1'][i].reshape(1, _H),
                       p['Wc2'][i].reshape(1, _H))
        ph, px = scat(ef, tr, row1, zh, zx)
        if i < _NLAYERS - 1:
            hh, coord, prow, pcol = nodek(
                hh, ph[0], ph[1], px[0], px[1], coord, temb,
                wn1[i, :_H], wn1[i, _H:], p['bn1'][i].reshape(1, _H),
                p['Wn2'][i], p['bn2'][i].reshape(1, _H),
                we1[i + 1, :_H], we1[i + 1, _H:2 * _H],
                p['be1'][i + 1].reshape(1, _H))
        else:
            h_out, coord = finalk(
                hh, ph[0], ph[1], px[0], px[1], coord,
                wn1[i, :_H], wn1[i, _H:], p['bn1'][i].reshape(1, _H),
                p['Wn2'][i], p['bn2'][i].reshape(1, _H),
                p['W_out'], p['b_out'].reshape(1, _H))

    return h_out, coord[:, :3]
